# Initial kernel scaffold; baseline (speedup 1.0000x reference)
import jax
import jax.numpy as jnp
from jax.experimental import pallas as pl


def kernel(node_features, edge_index, edge_features, memory, W1, b1, W2, b2, Wih, bih, Whh, bhh, Wemb, bemb):
    raise NotImplementedError("write your pallas kernel here")


# speedup vs baseline: 6.0792x; 6.0792x over previous
import jax
import jax.numpy as jnp
from jax.experimental import pallas as pl
from jax.experimental.pallas import tpu as pltpu

N = 10000
E = 320000
NPAD = 10240
BLK = 512


def _dense_body(mem_ref, memcol_ref, nf_ref, ef_ref, win_ref,
                w1a_ref, w1b_ref, w1c_ref, b1_ref, w2_ref, b2_ref,
                wih_ref, bih_ref, whh_ref, bhh_ref,
                wemba_ref, wembb_ref, bemb_ref,
                emb_out, mem_out):
    mem = mem_ref[...]
    memcol = memcol_ref[...]
    ef = ef_ref[...]

    x1 = (jnp.dot(mem, w1a_ref[...], preferred_element_type=jnp.float32)
          + jnp.dot(memcol, w1b_ref[...], preferred_element_type=jnp.float32)
          + jnp.dot(ef, w1c_ref[...], preferred_element_type=jnp.float32)
          + b1_ref[...])
    h1 = jnp.maximum(x1, 0.0)
    msg = jnp.dot(h1, w2_ref[...], preferred_element_type=jnp.float32) + b2_ref[...]

    gi = jnp.dot(msg, wih_ref[...], preferred_element_type=jnp.float32) + bih_ref[...]
    gh = jnp.dot(mem, whh_ref[...], preferred_element_type=jnp.float32) + bhh_ref[...]
    r = jax.nn.sigmoid(gi[:, :128] + gh[:, :128])
    z = jax.nn.sigmoid(gi[:, 128:256] + gh[:, 128:256])
    n = jnp.tanh(gi[:, 256:] + r * gh[:, 256:])
    gru = (1.0 - z) * n + z * mem

    mask = win_ref[...] >= 0
    newmem = jnp.where(mask, gru, mem)

    emb = (jnp.dot(newmem, wemba_ref[...], preferred_element_type=jnp.float32)
           + jnp.dot(nf_ref[...], wembb_ref[...], preferred_element_type=jnp.float32)
           + bemb_ref[...])
    emb_out[...] = emb
    mem_out[...] = newmem


def kernel(node_features, edge_index, edge_features, memory,
           W1, b1, W2, b2, Wih, bih, Whh, bhh, Wemb, bemb):
    row = edge_index[0]
    col = edge_index[1]

    winner = jnp.full((N,), -1, jnp.int32).at[row].max(
        jnp.arange(E, dtype=jnp.int32))
    e_safe = jnp.maximum(winner, 0)
    colw = col[e_safe]
    efw = edge_features[e_safe]
    memcol = memory[colw]

    pad_n = NPAD - N
    mem_p = jnp.pad(memory, ((0, pad_n), (0, 0)))
    nf_p = jnp.pad(node_features, ((0, pad_n), (0, 0)))
    memcol_p = jnp.pad(memcol, ((0, pad_n), (0, 0)))
    efw_p = jnp.pad(efw, ((0, pad_n), (0, 0)))
    win_p = jnp.pad(winner, (0, pad_n), constant_values=-1).reshape(NPAD, 1)

    grid = NPAD // BLK
    row_spec = lambda w: pl.BlockSpec((BLK, w), lambda i: (i, 0))
    full_spec = lambda a, b: pl.BlockSpec((a, b), lambda i: (0, 0))

    emb, newmem = pl.pallas_call(
        _dense_body,
        grid=(grid,),
        in_specs=[
            row_spec(128), row_spec(128), row_spec(128), row_spec(16),
            row_spec(1),
            full_spec(128, 128), full_spec(128, 128), full_spec(16, 128),
            full_spec(1, 128),
            full_spec(128, 128), full_spec(1, 128),
            full_spec(128, 384), full_spec(1, 384),
            full_spec(128, 384), full_spec(1, 384),
            full_spec(128, 128), full_spec(128, 128), full_spec(1, 128),
        ],
        out_specs=[row_spec(128), row_spec(128)],
        out_shape=[
            jax.ShapeDtypeStruct((NPAD, 128), jnp.float32),
            jax.ShapeDtypeStruct((NPAD, 128), jnp.float32),
        ],
    )(
        mem_p, memcol_p, nf_p, efw_p, win_p,
        W1[:, :128].T, W1[:, 128:256].T, W1[:, 256:].T, b1.reshape(1, 128),
        W2.T, b2.reshape(1, 128),
        Wih.T, bih.reshape(1, 384),
        Whh.T, bhh.reshape(1, 384),
        Wemb[:, :128].T, Wemb[:, 128:].T, bemb.reshape(1, 128),
    )
    return emb[:N], newmem[:N]


# speedup vs baseline: 8.9362x; 1.4700x over previous
import functools

import jax
import jax.numpy as jnp
from jax import lax
from jax.experimental import pallas as pl
from jax.experimental.pallas import tpu as pltpu
from jax.experimental.pallas import tpu_sc as plsc

N = 10000
E = 320000
NPAD = 12288
BLK = 512

NC = 2
NS = 16
L = 16
NH = NPAD // NC
EW = E // NS
NW = NH // NS
GCH = 64


def _sc_body(row_hbm, col_hbm, ef_hbm, mem_hbm,
             win_out, memcol_out, efw_out,
             rows_v, winner_v, shared, mbuf, wslice, eidx, colbuf,
             membuf, efbuf, sem):
    c = lax.axis_index("c")
    s = lax.axis_index("s")
    node_base = c * NH
    edge_base = s * EW
    lanes = lax.iota(jnp.int32, L)
    neg1 = jnp.full((L,), -1, jnp.int32)

    def init_body(i, _):
        winner_v[pl.ds(i * L, L)] = neg1
        return 0
    lax.fori_loop(0, NH // L, init_body, 0)

    pltpu.sync_copy(row_hbm.at[pl.ds(edge_base, EW)], rows_v)

    def scan_body(i, _):
        r = rows_v[pl.ds(i * L, L)]
        lidx = r - node_base
        inb = (lidx >= 0) & (lidx < NH)
        idxc = jnp.where(inb, lidx, 0)
        cur = plsc.load_gather(winner_v, [idxc])
        val = edge_base + i * L + lanes
        newv = jnp.maximum(cur, val)
        plsc.store_scatter(winner_v, [idxc], newv, mask=inb)
        return 0
    lax.fori_loop(0, EW // L, scan_body, 0)

    pltpu.sync_copy(winner_v, shared.at[s])
    plsc.subcore_barrier()
    pltpu.sync_copy(shared.at[:, pl.ds(s * NW, NW)], mbuf)

    def merge_body(k, _):
        acc = neg1
        for j in range(NS):
            acc = jnp.maximum(acc, mbuf[j, pl.ds(k * L, L)])
        wslice[pl.ds(k * L, L)] = acc
        eidx[pl.ds(k * L, L)] = jnp.maximum(acc, 0)
        return 0
    lax.fori_loop(0, NW // L, merge_body, 0)

    out_base = node_base + s * NW
    pltpu.sync_copy(wslice, win_out.at[pl.ds(out_base, NW)])

    for j in range(NW // GCH):
        idx_ch = eidx.at[pl.ds(j * GCH, GCH)]
        pltpu.async_copy(col_hbm.at[idx_ch], colbuf.at[pl.ds(j * GCH, GCH)],
                         sem).wait()
        pltpu.async_copy(ef_hbm.at[idx_ch], efbuf, sem).wait()
        pltpu.sync_copy(efbuf, efw_out.at[pl.ds(out_base + j * GCH, GCH)])
        pltpu.async_copy(mem_hbm.at[colbuf.at[pl.ds(j * GCH, GCH)]],
                         membuf, sem).wait()
        pltpu.sync_copy(membuf, memcol_out.at[pl.ds(out_base + j * GCH, GCH)])


_sc_gather = functools.partial(
    pl.kernel,
    out_type=[
        jax.ShapeDtypeStruct((NPAD,), jnp.int32),
        jax.ShapeDtypeStruct((NPAD, 128), jnp.float32),
        jax.ShapeDtypeStruct((NPAD, 16), jnp.float32),
    ],
    mesh=plsc.VectorSubcoreMesh(core_axis_name="c", subcore_axis_name="s"),
    scratch_types=[
        pltpu.VMEM((EW,), jnp.int32),
        pltpu.VMEM((NH,), jnp.int32),
        pltpu.VMEM_SHARED((NS, NH), jnp.int32),
        pltpu.VMEM((NS, NW), jnp.int32),
        pltpu.VMEM((NW,), jnp.int32),
        pltpu.VMEM((NW,), jnp.int32),
        pltpu.VMEM((NW,), jnp.int32),
        pltpu.VMEM((GCH, 128), jnp.float32),
        pltpu.VMEM((GCH, 16), jnp.float32),
        pltpu.SemaphoreType.DMA,
    ],
    compiler_params=pltpu.CompilerParams(needs_layout_passes=False,
                                         use_tc_tiling_on_sc=False),
)(_sc_body)


def _dense_body(mem_ref, memcol_ref, nf_ref, ef_ref, win_ref,
                w1a_ref, w1b_ref, w1c_ref, b1_ref, w2_ref, b2_ref,
                wih_ref, bih_ref, whh_ref, bhh_ref,
                wemba_ref, wembb_ref, bemb_ref,
                emb_out, mem_out):
    mem = mem_ref[...]
    memcol = memcol_ref[...]
    ef = ef_ref[...]

    x1 = (jnp.dot(mem, w1a_ref[...], preferred_element_type=jnp.float32)
          + jnp.dot(memcol, w1b_ref[...], preferred_element_type=jnp.float32)
          + jnp.dot(ef, w1c_ref[...], preferred_element_type=jnp.float32)
          + b1_ref[...])
    h1 = jnp.maximum(x1, 0.0)
    msg = jnp.dot(h1, w2_ref[...], preferred_element_type=jnp.float32) + b2_ref[...]

    gi = jnp.dot(msg, wih_ref[...], preferred_element_type=jnp.float32) + bih_ref[...]
    gh = jnp.dot(mem, whh_ref[...], preferred_element_type=jnp.float32) + bhh_ref[...]
    r = jax.nn.sigmoid(gi[:, :128] + gh[:, :128])
    z = jax.nn.sigmoid(gi[:, 128:256] + gh[:, 128:256])
    n = jnp.tanh(gi[:, 256:] + r * gh[:, 256:])
    gru = (1.0 - z) * n + z * mem

    mask = win_ref[...] >= 0
    newmem = jnp.where(mask, gru, mem)

    emb = (jnp.dot(newmem, wemba_ref[...], preferred_element_type=jnp.float32)
           + jnp.dot(nf_ref[...], wembb_ref[...], preferred_element_type=jnp.float32)
           + bemb_ref[...])
    emb_out[...] = emb
    mem_out[...] = newmem


def kernel(node_features, edge_index, edge_features, memory,
           W1, b1, W2, b2, Wih, bih, Whh, bhh, Wemb, bemb):
    row = edge_index[0]
    col = edge_index[1]

    win_p, memcol_p, efw_p = _sc_gather(row, col, edge_features, memory)
    win_p = win_p.reshape(NPAD, 1)

    pad_n = NPAD - N
    mem_p = jnp.pad(memory, ((0, pad_n), (0, 0)))
    nf_p = jnp.pad(node_features, ((0, pad_n), (0, 0)))

    grid = NPAD // BLK
    row_spec = lambda w: pl.BlockSpec((BLK, w), lambda i: (i, 0))
    full_spec = lambda a, b: pl.BlockSpec((a, b), lambda i: (0, 0))

    emb, newmem = pl.pallas_call(
        _dense_body,
        grid=(grid,),
        in_specs=[
            row_spec(128), row_spec(128), row_spec(128), row_spec(16),
            row_spec(1),
            full_spec(128, 128), full_spec(128, 128), full_spec(16, 128),
            full_spec(1, 128),
            full_spec(128, 128), full_spec(1, 128),
            full_spec(128, 384), full_spec(1, 384),
            full_spec(128, 384), full_spec(1, 384),
            full_spec(128, 128), full_spec(128, 128), full_spec(1, 128),
        ],
        out_specs=[row_spec(128), row_spec(128)],
        out_shape=[
            jax.ShapeDtypeStruct((NPAD, 128), jnp.float32),
            jax.ShapeDtypeStruct((NPAD, 128), jnp.float32),
        ],
    )(
        mem_p, memcol_p, nf_p, efw_p, win_p,
        W1[:, :128].T, W1[:, 128:256].T, W1[:, 256:].T, b1.reshape(1, 128),
        W2.T, b2.reshape(1, 128),
        Wih.T, bih.reshape(1, 384),
        Whh.T, bhh.reshape(1, 384),
        Wemb[:, :128].T, Wemb[:, 128:].T, bemb.reshape(1, 128),
    )
    return emb[:N], newmem[:N]


# speedup vs baseline: 9.5165x; 1.0649x over previous
import functools

import jax
import jax.numpy as jnp
from jax import lax
from jax.experimental import pallas as pl
from jax.experimental.pallas import tpu as pltpu
from jax.experimental.pallas import tpu_sc as plsc

N = 10000
E = 320000
NPAD = 12288
BLK = 400

NC = 2
NS = 16
L = 16
NH = NPAD // NC
EW = E // NS
NW = NH // NS
GCH = 128


def _sc_body(row_hbm, col_hbm, ef_hbm, mem_hbm,
             win_out, memcol_out, efw_out,
             rows_v, winner_v, shared, mbuf, wslice, eidx, colbuf,
             membuf, efbuf, sem, rsem):
    c = lax.axis_index("c")
    s = lax.axis_index("s")
    node_base = c * NH
    edge_base = s * EW
    lanes = lax.iota(jnp.int32, L)
    neg1 = jnp.full((L,), -1, jnp.int32)
    dump = jnp.full((L,), NH, jnp.int32) + lanes

    rows_cp = pltpu.async_copy(row_hbm.at[pl.ds(edge_base, EW)], rows_v, rsem)

    def init_body(i, _):
        winner_v[pl.ds(i * L, L)] = neg1
        return 0
    lax.fori_loop(0, (NH + L) // L, init_body, 0)
    rows_cp.wait()

    def scan_body(i, val):
        r = rows_v[pl.ds(i * L, L)]
        lidx = plsc.bitcast(r - node_base, jnp.uint32)
        idxc = plsc.bitcast(jnp.minimum(lidx, plsc.bitcast(dump, jnp.uint32)),
                            jnp.int32)
        plsc.store_scatter(winner_v, [idxc], val)
        return val + L
    lax.fori_loop(0, EW // L, scan_body, edge_base + lanes)

    pltpu.sync_copy(winner_v.at[pl.ds(0, NH)], shared.at[s])
    plsc.subcore_barrier()
    pltpu.sync_copy(shared.at[:, pl.ds(s * NW, NW)], mbuf)

    def merge_body(k, _):
        acc = neg1
        for j in range(NS):
            acc = jnp.maximum(acc, mbuf[j, pl.ds(k * L, L)])
        wslice[pl.ds(k * L, L)] = acc
        eidx[pl.ds(k * L, L)] = jnp.maximum(acc, 0)
        return 0
    lax.fori_loop(0, NW // L, merge_body, 0)

    out_base = node_base + s * NW
    win_cp = pltpu.async_copy(wslice, win_out.at[pl.ds(out_base, NW)], rsem)

    nch = NW // GCH
    col_cps = [
        pltpu.async_copy(col_hbm.at[eidx.at[pl.ds(j * GCH, GCH)]],
                         colbuf.at[pl.ds(j * GCH, GCH)], sem)
        for j in range(nch)
    ]
    for cp in col_cps:
        cp.wait()
    gather_cps = [
        pltpu.async_copy(ef_hbm.at[eidx.at[pl.ds(j * GCH, GCH)]],
                         efbuf.at[pl.ds(j * GCH, GCH)], sem)
        for j in range(nch)
    ] + [
        pltpu.async_copy(mem_hbm.at[colbuf.at[pl.ds(j * GCH, GCH)]],
                         membuf.at[pl.ds(j * GCH, GCH)], sem)
        for j in range(nch)
    ]
    for cp in gather_cps:
        cp.wait()
    pltpu.sync_copy(efbuf, efw_out.at[pl.ds(out_base, NW)])
    pltpu.sync_copy(membuf, memcol_out.at[pl.ds(out_base, NW)])
    win_cp.wait()


_sc_gather = functools.partial(
    pl.kernel,
    out_type=[
        jax.ShapeDtypeStruct((NPAD,), jnp.int32),
        jax.ShapeDtypeStruct((NPAD, 128), jnp.float32),
        jax.ShapeDtypeStruct((NPAD, 16), jnp.float32),
    ],
    mesh=plsc.VectorSubcoreMesh(core_axis_name="c", subcore_axis_name="s"),
    scratch_types=[
        pltpu.VMEM((EW,), jnp.int32),
        pltpu.VMEM((NH + L,), jnp.int32),
        pltpu.VMEM_SHARED((NS, NH), jnp.int32),
        pltpu.VMEM((NS, NW), jnp.int32),
        pltpu.VMEM((NW,), jnp.int32),
        pltpu.VMEM((NW,), jnp.int32),
        pltpu.VMEM((NW,), jnp.int32),
        pltpu.VMEM((NW, 128), jnp.float32),
        pltpu.VMEM((NW, 16), jnp.float32),
        pltpu.SemaphoreType.DMA,
        pltpu.SemaphoreType.DMA,
    ],
    compiler_params=pltpu.CompilerParams(needs_layout_passes=False,
                                         use_tc_tiling_on_sc=False),
)(_sc_body)


def _dense_body(mem_ref, memcol_ref, nf_ref, ef_ref, win_ref,
                w1a_ref, w1b_ref, w1c_ref, b1_ref, w2_ref, b2_ref,
                wih_ref, bih_ref, whh_ref, bhh_ref,
                wemba_ref, wembb_ref, bemb_ref,
                emb_out, mem_out):
    mem = mem_ref[...]
    memcol = memcol_ref[...]
    ef = ef_ref[...]

    x1 = (jnp.dot(mem, w1a_ref[...], preferred_element_type=jnp.float32)
          + jnp.dot(memcol, w1b_ref[...], preferred_element_type=jnp.float32)
          + jnp.dot(ef, w1c_ref[...], preferred_element_type=jnp.float32)
          + b1_ref[...])
    h1 = jnp.maximum(x1, 0.0)
    msg = jnp.dot(h1, w2_ref[...], preferred_element_type=jnp.float32) + b2_ref[...]

    gi = jnp.dot(msg, wih_ref[...], preferred_element_type=jnp.float32) + bih_ref[...]
    gh = jnp.dot(mem, whh_ref[...], preferred_element_type=jnp.float32) + bhh_ref[...]
    r = jax.nn.sigmoid(gi[:, :128] + gh[:, :128])
    z = jax.nn.sigmoid(gi[:, 128:256] + gh[:, 128:256])
    n = jnp.tanh(gi[:, 256:] + r * gh[:, 256:])
    gru = (1.0 - z) * n + z * mem

    mask = win_ref[...] >= 0
    newmem = jnp.where(mask, gru, mem)

    emb = (jnp.dot(newmem, wemba_ref[...], preferred_element_type=jnp.float32)
           + jnp.dot(nf_ref[...], wembb_ref[...], preferred_element_type=jnp.float32)
           + bemb_ref[...])
    emb_out[...] = emb
    mem_out[...] = newmem


def kernel(node_features, edge_index, edge_features, memory,
           W1, b1, W2, b2, Wih, bih, Whh, bhh, Wemb, bemb):
    row = edge_index[0]
    col = edge_index[1]

    win_p, memcol_p, efw_p = _sc_gather(row, col, edge_features, memory)
    win_p = win_p.reshape(NPAD, 1)

    grid = N // BLK
    row_spec = lambda w: pl.BlockSpec((BLK, w), lambda i: (i, 0))
    full_spec = lambda a, b: pl.BlockSpec((a, b), lambda i: (0, 0))

    emb, newmem = pl.pallas_call(
        _dense_body,
        grid=(grid,),
        in_specs=[
            row_spec(128), row_spec(128), row_spec(128), row_spec(16),
            row_spec(1),
            full_spec(128, 128), full_spec(128, 128), full_spec(16, 128),
            full_spec(1, 128),
            full_spec(128, 128), full_spec(1, 128),
            full_spec(128, 384), full_spec(1, 384),
            full_spec(128, 384), full_spec(1, 384),
            full_spec(128, 128), full_spec(128, 128), full_spec(1, 128),
        ],
        out_specs=[row_spec(128), row_spec(128)],
        out_shape=[
            jax.ShapeDtypeStruct((N, 128), jnp.float32),
            jax.ShapeDtypeStruct((N, 128), jnp.float32),
        ],
    )(
        memory, memcol_p, node_features, efw_p, win_p,
        W1[:, :128].T, W1[:, 128:256].T, W1[:, 256:].T, b1.reshape(1, 128),
        W2.T, b2.reshape(1, 128),
        Wih.T, bih.reshape(1, 384),
        Whh.T, bhh.reshape(1, 384),
        Wemb[:, :128].T, Wemb[:, 128:].T, bemb.reshape(1, 128),
    )
    return emb, newmem


# speedup vs baseline: 9.5764x; 1.0063x over previous
import functools

import jax
import jax.numpy as jnp
from jax import lax
from jax.experimental import pallas as pl
from jax.experimental.pallas import tpu as pltpu
from jax.experimental.pallas import tpu_sc as plsc

N = 10000
E = 320000
NPAD = 12288
BLK = 400

NC = 2
NS = 16
L = 16
NH = NPAD // NC
EW = E // NS
NW = NH // NS
GCH = 128


def _sc_body(ei_hbm, ef_hbm, mem_hbm,
             win_out, memcol_out, efw_out,
             rows_v, winner_v, shared, mbuf, wslice, eidx, colbuf,
             membuf, efbuf, sem, rsem):
    c = lax.axis_index("c")
    s = lax.axis_index("s")
    node_base = c * NH
    edge_base = s * EW
    lanes = lax.iota(jnp.int32, L)
    neg1 = jnp.full((L,), -1, jnp.int32)
    dump = jnp.full((L,), NH, jnp.int32) + lanes

    rows_cp = pltpu.async_copy(ei_hbm.at[0, pl.ds(edge_base, EW)], rows_v,
                               rsem)

    def init_body(i, _):
        winner_v[pl.ds(i * L, L)] = neg1
        return 0
    lax.fori_loop(0, (NH + L) // L, init_body, 0)
    rows_cp.wait()

    with jax.named_scope("p1_scan"):
        def scan_body(i, val):
            r = rows_v[pl.ds(i * L, L)]
            lidx = plsc.bitcast(r - node_base, jnp.uint32)
            idxc = plsc.bitcast(
                jnp.minimum(lidx, plsc.bitcast(dump, jnp.uint32)), jnp.int32)
            plsc.store_scatter(winner_v, [idxc], val)
            return val + L
        lax.fori_loop(0, EW // L, scan_body, edge_base + lanes)

    with jax.named_scope("p2_merge"):
        pltpu.sync_copy(winner_v.at[pl.ds(0, NH)], shared.at[s])
        plsc.subcore_barrier()
        pltpu.sync_copy(shared.at[:, pl.ds(s * NW, NW)], mbuf)

        def merge_body(k, _):
            acc = neg1
            for j in range(NS):
                acc = jnp.maximum(acc, mbuf[j, pl.ds(k * L, L)])
            wslice[pl.ds(k * L, L)] = acc
            eidx[pl.ds(k * L, L)] = jnp.maximum(acc, 0)
            return 0
        lax.fori_loop(0, NW // L, merge_body, 0)

    out_base = node_base + s * NW
    win_cp = pltpu.async_copy(wslice, win_out.at[pl.ds(out_base, NW)], rsem)

    with jax.named_scope("p3_gather"):
        nch = NW // GCH
        col_hbm = ei_hbm.at[1]
        col_cps = [
            pltpu.async_copy(col_hbm.at[eidx.at[pl.ds(j * GCH, GCH)]],
                             colbuf.at[pl.ds(j * GCH, GCH)], sem)
            for j in range(nch)
        ]
        ef_cps = [
            pltpu.async_copy(ef_hbm.at[eidx.at[pl.ds(j * GCH, GCH)]],
                             efbuf.at[pl.ds(j * GCH, GCH)], sem)
            for j in range(nch)
        ]
        for cp in col_cps:
            cp.wait()
        mem_cps = [
            pltpu.async_copy(mem_hbm.at[colbuf.at[pl.ds(j * GCH, GCH)]],
                             membuf.at[pl.ds(j * GCH, GCH)], sem)
            for j in range(nch)
        ]
        for cp in ef_cps:
            cp.wait()
        pltpu.sync_copy(efbuf, efw_out.at[pl.ds(out_base, NW)])
        for cp in mem_cps:
            cp.wait()
        pltpu.sync_copy(membuf, memcol_out.at[pl.ds(out_base, NW)])
    win_cp.wait()


_sc_gather = functools.partial(
    pl.kernel,
    out_type=[
        jax.ShapeDtypeStruct((NPAD,), jnp.int32),
        jax.ShapeDtypeStruct((NPAD, 128), jnp.float32),
        jax.ShapeDtypeStruct((NPAD, 16), jnp.float32),
    ],
    mesh=plsc.VectorSubcoreMesh(core_axis_name="c", subcore_axis_name="s"),
    scratch_types=[
        pltpu.VMEM((EW,), jnp.int32),
        pltpu.VMEM((NH + L,), jnp.int32),
        pltpu.VMEM_SHARED((NS, NH), jnp.int32),
        pltpu.VMEM((NS, NW), jnp.int32),
        pltpu.VMEM((NW,), jnp.int32),
        pltpu.VMEM((NW,), jnp.int32),
        pltpu.VMEM((NW,), jnp.int32),
        pltpu.VMEM((NW, 128), jnp.float32),
        pltpu.VMEM((NW, 16), jnp.float32),
        pltpu.SemaphoreType.DMA,
        pltpu.SemaphoreType.DMA,
    ],
    compiler_params=pltpu.CompilerParams(needs_layout_passes=False,
                                         use_tc_tiling_on_sc=False),
)(_sc_body)


def _dense_body(mem_ref, memcol_ref, nf_ref, ef_ref, win_ref,
                w1a_ref, w1b_ref, w1c_ref, b1_ref, w2_ref, b2_ref,
                wih_ref, bih_ref, whh_ref, bhh_ref,
                wemba_ref, wembb_ref, bemb_ref,
                emb_out, mem_out):
    mem = mem_ref[...]
    memcol = memcol_ref[...]
    ef = ef_ref[...]

    x1 = (jnp.dot(mem, w1a_ref[...], preferred_element_type=jnp.float32)
          + jnp.dot(memcol, w1b_ref[...], preferred_element_type=jnp.float32)
          + jnp.dot(ef, w1c_ref[...], preferred_element_type=jnp.float32)
          + b1_ref[...])
    h1 = jnp.maximum(x1, 0.0)
    msg = jnp.dot(h1, w2_ref[...], preferred_element_type=jnp.float32) + b2_ref[...]

    gi = jnp.dot(msg, wih_ref[...], preferred_element_type=jnp.float32) + bih_ref[...]
    gh = jnp.dot(mem, whh_ref[...], preferred_element_type=jnp.float32) + bhh_ref[...]
    r = jax.nn.sigmoid(gi[:, :128] + gh[:, :128])
    z = jax.nn.sigmoid(gi[:, 128:256] + gh[:, 128:256])
    n = jnp.tanh(gi[:, 256:] + r * gh[:, 256:])
    gru = (1.0 - z) * n + z * mem

    mask = win_ref[...] >= 0
    newmem = jnp.where(mask, gru, mem)

    emb = (jnp.dot(newmem, wemba_ref[...], preferred_element_type=jnp.float32)
           + jnp.dot(nf_ref[...], wembb_ref[...], preferred_element_type=jnp.float32)
           + bemb_ref[...])
    emb_out[...] = emb
    mem_out[...] = newmem


def kernel(node_features, edge_index, edge_features, memory,
           W1, b1, W2, b2, Wih, bih, Whh, bhh, Wemb, bemb):
    win_p, memcol_p, efw_p = _sc_gather(edge_index, edge_features, memory)
    win_p = win_p.reshape(NPAD, 1)

    grid = N // BLK
    row_spec = lambda w: pl.BlockSpec((BLK, w), lambda i: (i, 0))
    full_spec = lambda a, b: pl.BlockSpec((a, b), lambda i: (0, 0))

    emb, newmem = pl.pallas_call(
        _dense_body,
        grid=(grid,),
        in_specs=[
            row_spec(128), row_spec(128), row_spec(128), row_spec(16),
            row_spec(1),
            full_spec(128, 128), full_spec(128, 128), full_spec(16, 128),
            full_spec(1, 128),
            full_spec(128, 128), full_spec(1, 128),
            full_spec(128, 384), full_spec(1, 384),
            full_spec(128, 384), full_spec(1, 384),
            full_spec(128, 128), full_spec(128, 128), full_spec(1, 128),
        ],
        out_specs=[row_spec(128), row_spec(128)],
        out_shape=[
            jax.ShapeDtypeStruct((N, 128), jnp.float32),
            jax.ShapeDtypeStruct((N, 128), jnp.float32),
        ],
    )(
        memory, memcol_p, node_features, efw_p, win_p,
        W1[:, :128].T, W1[:, 128:256].T, W1[:, 256:].T, b1.reshape(1, 128),
        W2.T, b2.reshape(1, 128),
        Wih.T, bih.reshape(1, 384),
        Whh.T, bhh.reshape(1, 384),
        Wemb[:, :128].T, Wemb[:, 128:].T, bemb.reshape(1, 128),
    )
    return emb, newmem


# speedup vs baseline: 14.2568x; 1.4887x over previous
import functools

import jax
import jax.numpy as jnp
from jax import lax
from jax.experimental import pallas as pl
from jax.experimental.pallas import tpu as pltpu
from jax.experimental.pallas import tpu_sc as plsc

N = 10000
E = 320000
NPAD = 12288
BLK = 400

NC = 2
NS = 16
L = 16
NH = NPAD // NC
EW = E // NS
NW = NH // NS
GCH = 128


def _sc_body(ei_hbm, ef_hbm, mem_hbm,
             win_out, memcol_out, efw_out,
             rows_v, winner_v, shared, mbuf, wslice, eidx, colbuf,
             membuf, efbuf, sem, rsem):
    c = lax.axis_index("c")
    s = lax.axis_index("s")
    node_base = c * NH
    edge_base = s * EW
    lanes = lax.iota(jnp.int32, L)
    neg1 = jnp.full((L,), -1, jnp.int32)
    dump = jnp.full((L,), NH, jnp.int32) + lanes

    rows_cp = pltpu.async_copy(ei_hbm.at[0, pl.ds(edge_base, EW)], rows_v,
                               rsem)

    def init_body(i, _):
        winner_v[pl.ds(i * L, L)] = neg1
        return 0
    lax.fori_loop(0, (NH + L) // L, init_body, 0)
    rows_cp.wait()

    with jax.named_scope("p1_scan"):
        def scan_body(i, val):
            r = rows_v[pl.ds(i * L, L)]
            lidx = plsc.bitcast(r - node_base, jnp.uint32)
            idxc = plsc.bitcast(
                jnp.minimum(lidx, plsc.bitcast(dump, jnp.uint32)), jnp.int32)
            plsc.store_scatter(winner_v, [idxc], val)
            return val + L
        lax.fori_loop(0, EW // L, scan_body, edge_base + lanes)

    out_base = node_base + s * NW
    with jax.named_scope("p2_merge"):
        pltpu.sync_copy(winner_v.at[pl.ds(0, NH)], shared.at[s])
        plsc.subcore_barrier()
        pltpu.sync_copy(shared.at[:, pl.ds(s * NW, NW)], mbuf)

        def merge_body(k, _):
            acc = neg1
            for j in range(NS):
                acc = jnp.maximum(acc, mbuf[j, pl.ds(k * L, L)])
            wslice[pl.ds(k * L, L)] = acc
            dummy = out_base + k * L + lanes
            eidx[pl.ds(k * L, L)] = jnp.where(acc >= 0, acc, dummy)
            return 0
        lax.fori_loop(0, NW // L, merge_body, 0)
    win_cp = pltpu.async_copy(wslice, win_out.at[pl.ds(out_base, NW)], rsem)

    with jax.named_scope("p3_gather"):
        nch = NW // GCH
        col_hbm = ei_hbm.at[1]
        col_cps = [
            pltpu.async_copy(col_hbm.at[eidx.at[pl.ds(j * GCH, GCH)]],
                             colbuf.at[pl.ds(j * GCH, GCH)], sem)
            for j in range(nch)
        ]
        ef_cps = [
            pltpu.async_copy(ef_hbm.at[eidx.at[pl.ds(j * GCH, GCH)]],
                             efbuf.at[pl.ds(j * GCH, GCH)], sem)
            for j in range(nch)
        ]
        for cp in col_cps:
            cp.wait()
        mem_cps = [
            pltpu.async_copy(mem_hbm.at[colbuf.at[pl.ds(j * GCH, GCH)]],
                             membuf.at[pl.ds(j * GCH, GCH)], sem)
            for j in range(nch)
        ]
        for cp in ef_cps:
            cp.wait()
        pltpu.sync_copy(efbuf, efw_out.at[pl.ds(out_base, NW)])
        for cp in mem_cps:
            cp.wait()
        pltpu.sync_copy(membuf, memcol_out.at[pl.ds(out_base, NW)])
    win_cp.wait()


_sc_gather = functools.partial(
    pl.kernel,
    out_type=[
        jax.ShapeDtypeStruct((NPAD,), jnp.int32),
        jax.ShapeDtypeStruct((NPAD, 128), jnp.float32),
        jax.ShapeDtypeStruct((NPAD, 16), jnp.float32),
    ],
    mesh=plsc.VectorSubcoreMesh(core_axis_name="c", subcore_axis_name="s"),
    scratch_types=[
        pltpu.VMEM((EW,), jnp.int32),
        pltpu.VMEM((NH + L,), jnp.int32),
        pltpu.VMEM_SHARED((NS, NH), jnp.int32),
        pltpu.VMEM((NS, NW), jnp.int32),
        pltpu.VMEM((NW,), jnp.int32),
        pltpu.VMEM((NW,), jnp.int32),
        pltpu.VMEM((NW,), jnp.int32),
        pltpu.VMEM((NW, 128), jnp.float32),
        pltpu.VMEM((NW, 16), jnp.float32),
        pltpu.SemaphoreType.DMA,
        pltpu.SemaphoreType.DMA,
    ],
    compiler_params=pltpu.CompilerParams(needs_layout_passes=False,
                                         use_tc_tiling_on_sc=False),
)(_sc_body)


def _dense_body(mem_ref, memcol_ref, nf_ref, ef_ref, win_ref,
                w1a_ref, w1b_ref, w1c_ref, b1_ref, w2_ref, b2_ref,
                wih_ref, bih_ref, whh_ref, bhh_ref,
                wemba_ref, wembb_ref, bemb_ref,
                emb_out, mem_out):
    mem = mem_ref[...]
    memcol = memcol_ref[...]
    ef = ef_ref[...]

    x1 = (jnp.dot(mem, w1a_ref[...], preferred_element_type=jnp.float32)
          + jnp.dot(memcol, w1b_ref[...], preferred_element_type=jnp.float32)
          + jnp.dot(ef, w1c_ref[...], preferred_element_type=jnp.float32)
          + b1_ref[...])
    h1 = jnp.maximum(x1, 0.0)
    msg = jnp.dot(h1, w2_ref[...], preferred_element_type=jnp.float32) + b2_ref[...]

    gi = jnp.dot(msg, wih_ref[...], preferred_element_type=jnp.float32) + bih_ref[...]
    gh = jnp.dot(mem, whh_ref[...], preferred_element_type=jnp.float32) + bhh_ref[...]
    r = jax.nn.sigmoid(gi[:, :128] + gh[:, :128])
    z = jax.nn.sigmoid(gi[:, 128:256] + gh[:, 128:256])
    n = jnp.tanh(gi[:, 256:] + r * gh[:, 256:])
    gru = (1.0 - z) * n + z * mem

    mask = win_ref[...] >= 0
    newmem = jnp.where(mask, gru, mem)

    emb = (jnp.dot(newmem, wemba_ref[...], preferred_element_type=jnp.float32)
           + jnp.dot(nf_ref[...], wembb_ref[...], preferred_element_type=jnp.float32)
           + bemb_ref[...])
    emb_out[...] = emb
    mem_out[...] = newmem


def kernel(node_features, edge_index, edge_features, memory,
           W1, b1, W2, b2, Wih, bih, Whh, bhh, Wemb, bemb):
    win_p, memcol_p, efw_p = _sc_gather(edge_index, edge_features, memory)
    win_p = win_p.reshape(NPAD, 1)

    grid = N // BLK
    row_spec = lambda w: pl.BlockSpec((BLK, w), lambda i: (i, 0))
    full_spec = lambda a, b: pl.BlockSpec((a, b), lambda i: (0, 0))

    emb, newmem = pl.pallas_call(
        _dense_body,
        grid=(grid,),
        in_specs=[
            row_spec(128), row_spec(128), row_spec(128), row_spec(16),
            row_spec(1),
            full_spec(128, 128), full_spec(128, 128), full_spec(16, 128),
            full_spec(1, 128),
            full_spec(128, 128), full_spec(1, 128),
            full_spec(128, 384), full_spec(1, 384),
            full_spec(128, 384), full_spec(1, 384),
            full_spec(128, 128), full_spec(128, 128), full_spec(1, 128),
        ],
        out_specs=[row_spec(128), row_spec(128)],
        out_shape=[
            jax.ShapeDtypeStruct((N, 128), jnp.float32),
            jax.ShapeDtypeStruct((N, 128), jnp.float32),
        ],
    )(
        memory, memcol_p, node_features, efw_p, win_p,
        W1[:, :128].T, W1[:, 128:256].T, W1[:, 256:].T, b1.reshape(1, 128),
        W2.T, b2.reshape(1, 128),
        Wih.T, bih.reshape(1, 384),
        Whh.T, bhh.reshape(1, 384),
        Wemb[:, :128].T, Wemb[:, 128:].T, bemb.reshape(1, 128),
    )
    return emb, newmem
